# Initial kernel scaffold; baseline (speedup 1.0000x reference)
import jax
import jax.numpy as jnp
from jax.experimental import pallas as pl


def kernel(x, table, W_ih, W_hh, b_ih, b_hh, W1, b1, W2, b2):
    raise NotImplementedError("write your pallas kernel here")


# speedup vs baseline: 15.7185x; 15.7185x over previous
import functools

import jax
import jax.numpy as jnp
from jax import lax
from jax.experimental import pallas as pl
from jax.experimental.pallas import tpu as pltpu
from jax.experimental.pallas import tpu_sc as plsc

V = 1000000
E = 32
H = 64
C = 2
B = 4096
L = 200
H2 = H // 2

_NC = 2
_NS = 16
_NW = _NC * _NS

_GRP = 128
_NGRP = (L * B) // _GRP
_GRP_PER_W = _NGRP // _NW
_KGRP = 8
_NCHUNK = _GRP_PER_W // _KGRP


def _sc_gather(table, idx_grp):
    mesh = plsc.VectorSubcoreMesh(core_axis_name="c", subcore_axis_name="s")

    @functools.partial(
        pl.kernel,
        mesh=mesh,
        out_type=jax.ShapeDtypeStruct((_NGRP, _GRP, E), jnp.float32),
        scratch_types=[
            pltpu.VMEM((_KGRP, _GRP), jnp.int32),
            pltpu.VMEM((_KGRP, _GRP, E), jnp.float32),
            pltpu.SemaphoreType.DMA,
        ],
        compiler_params=pltpu.CompilerParams(use_tc_tiling_on_sc=False),
    )
    def gather_kernel(table_hbm, idx_hbm, out_hbm, idx_v, rows_v, sem):
        wid = lax.axis_index("s") * _NC + lax.axis_index("c")
        base = wid * _GRP_PER_W

        def chunk_body(i, carry):
            g0 = base + i * _KGRP
            pltpu.sync_copy(idx_hbm.at[pl.ds(g0, _KGRP)], idx_v)
            copies = []
            for j in range(_KGRP):
                copies.append(
                    pltpu.async_copy(table_hbm.at[idx_v.at[j]], rows_v.at[j], sem)
                )
            for cp in copies:
                cp.wait()
            pltpu.sync_copy(rows_v, out_hbm.at[pl.ds(g0, _KGRP)])
            return carry

        lax.fori_loop(0, _NCHUNK, chunk_body, 0)

    return gather_kernel(table, idx_grp)


def _lstm_body(emb_ref, wih_ref, whh_ref, b_ref, w1_ref, b1_ref, w2_ref,
               b2_ref, out_ref, h_ref, c_ref):
    l = pl.program_id(0)

    @pl.when(l == 0)
    def _():
        h_ref[...] = jnp.zeros_like(h_ref)
        c_ref[...] = jnp.zeros_like(c_ref)

    x_t = emb_ref[0]
    h = h_ref[...]
    c = c_ref[...]

    gates = jnp.dot(x_t, wih_ref[...], preferred_element_type=jnp.float32)
    gates = gates + jnp.dot(h, whh_ref[...], preferred_element_type=jnp.float32)
    gates = gates + b_ref[...]

    i_g = _sigmoid(gates[:, 0 * H:1 * H])
    f_g = _sigmoid(gates[:, 1 * H:2 * H])
    g_g = jnp.tanh(gates[:, 2 * H:3 * H])
    o_g = _sigmoid(gates[:, 3 * H:4 * H])
    c_new = f_g * c + i_g * g_g
    h_new = o_g * jnp.tanh(c_new)
    c_ref[...] = c_new
    h_ref[...] = h_new

    @pl.when(l == L - 1)
    def _():
        z = jnp.dot(h_new, w1_ref[...], preferred_element_type=jnp.float32)
        z = jnp.maximum(z + b1_ref[...], 0.0)
        out = jnp.dot(z, w2_ref[...], preferred_element_type=jnp.float32)
        out_ref[...] = out + b2_ref[...]


def _sigmoid(x):
    return 0.5 * (jnp.tanh(0.5 * x) + 1.0)


def _lstm_fc(emb, wihT, whhT, bias, w1T, b1, w2T, b2):
    return pl.pallas_call(
        _lstm_body,
        grid=(L,),
        in_specs=[
            pl.BlockSpec((1, B, E), lambda l: (l, 0, 0)),
            pl.BlockSpec((E, 4 * H), lambda l: (0, 0)),
            pl.BlockSpec((H, 4 * H), lambda l: (0, 0)),
            pl.BlockSpec((1, 4 * H), lambda l: (0, 0)),
            pl.BlockSpec((H, H2), lambda l: (0, 0)),
            pl.BlockSpec((1, H2), lambda l: (0, 0)),
            pl.BlockSpec((H2, C), lambda l: (0, 0)),
            pl.BlockSpec((1, C), lambda l: (0, 0)),
        ],
        out_specs=pl.BlockSpec((B, C), lambda l: (0, 0)),
        out_shape=jax.ShapeDtypeStruct((B, C), jnp.float32),
        scratch_shapes=[
            pltpu.VMEM((B, H), jnp.float32),
            pltpu.VMEM((B, H), jnp.float32),
        ],
    )(emb, wihT, whhT, bias, w1T, b1, w2T, b2)


def kernel(x, table, W_ih, W_hh, b_ih, b_hh, W1, b1, W2, b2):
    x = x.astype(jnp.int32)
    idx_grp = x.T.reshape(_NGRP, _GRP)
    emb = _sc_gather(table, idx_grp).reshape(L, B, E)
    out = _lstm_fc(
        emb,
        W_ih.T,
        W_hh.T,
        (b_ih + b_hh).reshape(1, 4 * H),
        W1.T,
        b1.reshape(1, H2),
        W2.T,
        b2.reshape(1, C),
    )
    return out[None]
